# baseline (speedup 1.0000x reference)
import jax
import jax.numpy as jnp
from jax import lax
from jax.experimental import pallas as pl
from jax.experimental.pallas import tpu as pltpu
from jax.experimental.pallas import tpu_sc as plsc

N = 10000
D = 256
DH = 128
E = 160000
NSUB = 16
EPT = E // NSUB
CH = 80
NCH = EPT // CH
GP = 25
NGP = NCH // GP
RQ = 624
TAIL = N - NSUB * RQ

BR = 2000
NB = N // BR

def _mesh():
    return plsc.VectorSubcoreMesh(core_axis_name="core",
                                  subcore_axis_name="subcore")


def _deg_body(dst_hbm, ones_hbm, zeros_hbm, out_hbm, dst_v, ones_v, acc):
    c = lax.axis_index("core")
    s = lax.axis_index("subcore")
    pltpu.sync_copy(dst_hbm.at[s], dst_v)
    pltpu.sync_copy(ones_hbm, ones_v)
    pltpu.sync_copy(zeros_hbm, acc.at[pl.ds(s * RQ, RQ)])

    @pl.when(s == 0)
    def _():
        pltpu.sync_copy(zeros_hbm.at[pl.ds(0, TAIL)],
                        acc.at[pl.ds(NSUB * RQ, TAIL)])

    plsc.subcore_barrier()
    off = c * 62

    @pl.loop(0, 62)
    def _(j):
        pltpu.sync_copy(ones_v, acc.at[dst_v.at[off + j]], add=True)

    @pl.when(c == 1)
    def _():
        pltpu.sync_copy(ones_v, acc.at[dst_v.at[124]], add=True)

    plsc.subcore_barrier()
    pltpu.sync_copy(acc.at[pl.ds(s * RQ, RQ)],
                    out_hbm.at[pl.ds(c * N + s * RQ, RQ)])

    @pl.when(s == 0)
    def _():
        pltpu.sync_copy(acc.at[pl.ds(NSUB * RQ, TAIL)],
                        out_hbm.at[pl.ds(c * N + NSUB * RQ, TAIL)])


def _agg_body(y_hbm, srcw_hbm, dst_hbm, out_hbm, src_v, dst_v, row0, row1,
              acc, sem0, sem1, ssem0, ssem1):
    c = lax.axis_index("core")
    s = lax.axis_index("subcore")
    pltpu.sync_copy(y_hbm.at[pl.ds(c * N + s * RQ, RQ)],
                    acc.at[pl.ds(s * RQ, RQ)])

    @pl.when(s == 0)
    def _():
        pltpu.sync_copy(y_hbm.at[pl.ds(c * N + NSUB * RQ, TAIL)],
                        acc.at[pl.ds(NSUB * RQ, TAIL)])

    plsc.subcore_barrier()

    @pl.loop(0, NGP)
    def _(g):
        pltpu.sync_copy(srcw_hbm.at[c, s, g], src_v)
        pltpu.sync_copy(dst_hbm.at[s, g], dst_v)
        pltpu.async_copy(y_hbm.at[src_v.at[0]], row0, sem0)
        pltpu.async_copy(y_hbm.at[src_v.at[1]], row1, sem1)

        @pl.loop(0, (GP - 1) // 2)
        def _(i):
            j = 2 * i
            pltpu.make_async_copy(y_hbm.at[src_v.at[j]], row0, sem0).wait()
            d0 = pltpu.async_copy(row0, acc.at[dst_v.at[j]], ssem0, add=True)
            pltpu.make_async_copy(y_hbm.at[src_v.at[j + 1]], row1, sem1).wait()
            d1 = pltpu.async_copy(row1, acc.at[dst_v.at[j + 1]], ssem1, add=True)
            d0.wait()
            pltpu.async_copy(y_hbm.at[src_v.at[j + 2]], row0, sem0)
            d1.wait()

            @pl.when(j + 3 < GP)
            def _():
                pltpu.async_copy(y_hbm.at[src_v.at[j + 3]], row1, sem1)

        pltpu.make_async_copy(y_hbm.at[src_v.at[GP - 1]], row0, sem0).wait()
        pltpu.sync_copy(row0, acc.at[dst_v.at[GP - 1]], add=True)

    plsc.subcore_barrier()
    pltpu.sync_copy(acc.at[pl.ds(s * RQ, RQ)],
                    out_hbm.at[pl.ds(c * N + s * RQ, RQ)])

    @pl.when(s == 0)
    def _():
        pltpu.sync_copy(acc.at[pl.ds(NSUB * RQ, TAIL)],
                        out_hbm.at[pl.ds(c * N + NSUB * RQ, TAIL)])


def _deg_call(dst_f, ones, zeros):
    fn = pl.kernel(
        _deg_body,
        out_type=jax.ShapeDtypeStruct((2 * N, DH), jnp.float32),
        mesh=_mesh(),
        scratch_types=[
            pltpu.VMEM((NCH, CH), jnp.int32),
            pltpu.VMEM((CH, DH), jnp.float32),
            pltpu.VMEM_SHARED((N, DH), jnp.float32),
        ],
    )
    return fn(dst_f, ones, zeros)


def _agg_call(y, srcw, dst_r):
    fn = pl.kernel(
        _agg_body,
        out_type=jax.ShapeDtypeStruct((2 * N, DH), jnp.float32),
        mesh=_mesh(),
        scratch_types=[
            pltpu.VMEM((GP, CH), jnp.int32),
            pltpu.VMEM((GP, CH), jnp.int32),
            pltpu.VMEM((CH, DH), jnp.float32),
            pltpu.VMEM((CH, DH), jnp.float32),
            pltpu.VMEM_SHARED((N, DH), jnp.float32),
            pltpu.SemaphoreType.DMA,
            pltpu.SemaphoreType.DMA,
            pltpu.SemaphoreType.DMA,
            pltpu.SemaphoreType.DMA,
        ],
    )
    return fn(y, srcw, dst_r)


def _dis_block(p0_ref, p1_ref):
    deg = 1.0 + p0_ref[:, :1] + p1_ref[:, :1]
    return lax.rsqrt(deg)


def _tca_body(x_ref, w_ref, o_ref):
    o_ref[...] = jnp.dot(x_ref[...], w_ref[...],
                         preferred_element_type=jnp.float32)


def _scale_body(u_ref, p0_ref, p1_ref, o_ref):
    o_ref[...] = u_ref[...] * _dis_block(p0_ref, p1_ref)


def _tcb_body(al_ref, ah_ref, p0_ref, p1_ref, w_ref, b_ref, o_ref):
    dis = _dis_block(p0_ref, p1_ref)
    agg = jnp.concatenate([al_ref[...], ah_ref[...]], axis=1)
    h = jnp.maximum(agg * dis + b_ref[...], 0.0)
    y = jnp.dot(h, w_ref[...], preferred_element_type=jnp.float32)
    o_ref[...] = y * dis


def _tcc_body(al_ref, ah_ref, p0_ref, p1_ref, b2_ref, wp1_ref, bp1_ref,
              wp2_ref, bp2_ref, o_ref):
    dis = _dis_block(p0_ref, p1_ref)
    agg = jnp.concatenate([al_ref[...], ah_ref[...]], axis=1)
    h = jnp.maximum(agg * dis + b2_ref[...], 0.0)
    t = jnp.dot(h, wp1_ref[...], preferred_element_type=jnp.float32) + bp1_ref[...]
    o = jnp.dot(t, wp2_ref[...], preferred_element_type=jnp.float32) + bp2_ref[...]
    m = jnp.max(o, axis=1, keepdims=True)
    ex = jnp.exp(o - m)
    ssum = jnp.sum(ex, axis=1, keepdims=True)
    o_ref[...] = (o - m) - jnp.log(ssum)


def _tca_call(x, W1):
    return pl.pallas_call(
        _tca_body,
        grid=(NB, 2),
        in_specs=[
            pl.BlockSpec((BR, D), lambda i, h: (i, 0)),
            pl.BlockSpec((D, DH), lambda i, h: (0, h)),
        ],
        out_specs=pl.BlockSpec((BR, DH), lambda i, h: (i + h * NB, 0)),
        out_shape=jax.ShapeDtypeStruct((2 * N, DH), jnp.float32),
    )(x, W1)


def _scale_call(u, p8):
    return pl.pallas_call(
        _scale_body,
        grid=(2 * NB,),
        in_specs=[
            pl.BlockSpec((BR, DH), lambda i: (i, 0)),
            pl.BlockSpec((BR, 8), lambda i: (i % NB, 0)),
            pl.BlockSpec((BR, 8), lambda i: (i % NB + NB, 0)),
        ],
        out_specs=pl.BlockSpec((BR, DH), lambda i: (i, 0)),
        out_shape=jax.ShapeDtypeStruct((2 * N, DH), jnp.float32),
    )(u, p8, p8)


def _tcb_call(agg, p8, W2, b1):
    return pl.pallas_call(
        _tcb_body,
        grid=(NB, 2),
        in_specs=[
            pl.BlockSpec((BR, DH), lambda i, h: (i, 0)),
            pl.BlockSpec((BR, DH), lambda i, h: (i + NB, 0)),
            pl.BlockSpec((BR, 8), lambda i, h: (i, 0)),
            pl.BlockSpec((BR, 8), lambda i, h: (i + NB, 0)),
            pl.BlockSpec((D, DH), lambda i, h: (0, h)),
            pl.BlockSpec((1, D), lambda i, h: (0, 0)),
        ],
        out_specs=pl.BlockSpec((BR, DH), lambda i, h: (i + h * NB, 0)),
        out_shape=jax.ShapeDtypeStruct((2 * N, DH), jnp.float32),
    )(agg, agg, p8, p8, W2, b1)


def _tcc_call(agg, p8, b2, Wp1, bp1, Wp2, bp2):
    return pl.pallas_call(
        _tcc_body,
        grid=(NB,),
        in_specs=[
            pl.BlockSpec((BR, DH), lambda i: (i, 0)),
            pl.BlockSpec((BR, DH), lambda i: (i + NB, 0)),
            pl.BlockSpec((BR, 8), lambda i: (i, 0)),
            pl.BlockSpec((BR, 8), lambda i: (i + NB, 0)),
            pl.BlockSpec((1, D), lambda i: (0, 0)),
            pl.BlockSpec((D, D), lambda i: (0, 0)),
            pl.BlockSpec((1, D), lambda i: (0, 0)),
            pl.BlockSpec((D, D), lambda i: (0, 0)),
            pl.BlockSpec((1, D), lambda i: (0, 0)),
        ],
        out_specs=pl.BlockSpec((BR, D), lambda i: (i, 0)),
        out_shape=jax.ShapeDtypeStruct((N, D), jnp.float32),
    )(agg, agg, p8, p8, b2, Wp1, bp1, Wp2, bp2)


def kernel(x, edge_index, W1, b1, W2, b2, Wp1, bp1, Wp2, bp2):
    ei = edge_index.astype(jnp.int32)
    src = ei[0]
    dst = ei[1]
    srcw = jnp.stack([src, src + N]).reshape(2, NSUB, NGP, GP, CH)
    dst_r = dst.reshape(NSUB, NGP, GP, CH)
    dst_f = dst.reshape(NSUB, NCH, CH)
    ones = jnp.ones((CH, DH), jnp.float32)
    zeros = jnp.zeros((RQ, DH), jnp.float32)

    degs = _deg_call(dst_f, ones, zeros)
    p8 = degs[:, :8]
    xwr = _tca_call(x, W1)
    y1 = _scale_call(xwr, p8)
    agg1 = _agg_call(y1, srcw, dst_r)
    y2 = _tcb_call(agg1, p8, W2, b1.reshape(1, D))
    agg2 = _agg_call(y2, srcw, dst_r)
    return _tcc_call(agg2, p8, b2.reshape(1, D), Wp1, bp1.reshape(1, D),
                     Wp2, bp2.reshape(1, D))


# speedup vs baseline: 1.1797x; 1.1797x over previous
import jax
import jax.numpy as jnp
from jax import lax
from jax.experimental import pallas as pl
from jax.experimental.pallas import tpu as pltpu
from jax.experimental.pallas import tpu_sc as plsc

N = 10000
D = 256
DH = 128
E = 160000
NSUB = 16
EPT = E // NSUB
CH = 80
NCH = EPT // CH
GP = 25
NGP = NCH // GP
RQ = 624
TAIL = N - NSUB * RQ

BR = 2000
NB = N // BR

def _mesh():
    return plsc.VectorSubcoreMesh(core_axis_name="core",
                                  subcore_axis_name="subcore")


def _deg_body(dst_hbm, ones_hbm, zeros_hbm, out_hbm, dst_v, ones_v, acc):
    c = lax.axis_index("core")
    s = lax.axis_index("subcore")
    pltpu.sync_copy(dst_hbm.at[s], dst_v)
    pltpu.sync_copy(ones_hbm, ones_v)
    pltpu.sync_copy(zeros_hbm, acc.at[pl.ds(s * RQ, RQ)])

    @pl.when(s == 0)
    def _():
        pltpu.sync_copy(zeros_hbm.at[pl.ds(0, TAIL)],
                        acc.at[pl.ds(NSUB * RQ, TAIL)])

    plsc.subcore_barrier()
    off = c * 62

    @pl.loop(0, 62)
    def _(j):
        pltpu.sync_copy(ones_v, acc.at[dst_v.at[off + j]], add=True)

    @pl.when(c == 1)
    def _():
        pltpu.sync_copy(ones_v, acc.at[dst_v.at[124]], add=True)

    plsc.subcore_barrier()
    pltpu.sync_copy(acc.at[pl.ds(s * RQ, RQ)],
                    out_hbm.at[pl.ds(c * N + s * RQ, RQ)])

    @pl.when(s == 0)
    def _():
        pltpu.sync_copy(acc.at[pl.ds(NSUB * RQ, TAIL)],
                        out_hbm.at[pl.ds(c * N + NSUB * RQ, TAIL)])


def _agg_body(y_hbm, srcw_hbm, dst_hbm, out_hbm, src_v, dst_v, row0, row1,
              acc, sem0, sem1):
    c = lax.axis_index("core")
    s = lax.axis_index("subcore")
    pltpu.sync_copy(y_hbm.at[pl.ds(c * N + s * RQ, RQ)],
                    acc.at[pl.ds(s * RQ, RQ)])

    @pl.when(s == 0)
    def _():
        pltpu.sync_copy(y_hbm.at[pl.ds(c * N + NSUB * RQ, TAIL)],
                        acc.at[pl.ds(NSUB * RQ, TAIL)])

    plsc.subcore_barrier()

    @pl.loop(0, NGP)
    def _(g):
        pltpu.sync_copy(srcw_hbm.at[c, s, g], src_v)
        pltpu.sync_copy(dst_hbm.at[s, g], dst_v)
        pltpu.async_copy(y_hbm.at[src_v.at[0]], row0, sem0)

        @pl.loop(0, (GP - 1) // 2)
        def _(i):
            j = 2 * i
            pltpu.async_copy(y_hbm.at[src_v.at[j + 1]], row1, sem1)
            pltpu.make_async_copy(y_hbm.at[src_v.at[j]], row0, sem0).wait()
            pltpu.sync_copy(row0, acc.at[dst_v.at[j]], add=True)
            pltpu.async_copy(y_hbm.at[src_v.at[j + 2]], row0, sem0)
            pltpu.make_async_copy(y_hbm.at[src_v.at[j + 1]], row1, sem1).wait()
            pltpu.sync_copy(row1, acc.at[dst_v.at[j + 1]], add=True)

        pltpu.make_async_copy(y_hbm.at[src_v.at[GP - 1]], row0, sem0).wait()
        pltpu.sync_copy(row0, acc.at[dst_v.at[GP - 1]], add=True)

    plsc.subcore_barrier()
    pltpu.sync_copy(acc.at[pl.ds(s * RQ, RQ)],
                    out_hbm.at[pl.ds(c * N + s * RQ, RQ)])

    @pl.when(s == 0)
    def _():
        pltpu.sync_copy(acc.at[pl.ds(NSUB * RQ, TAIL)],
                        out_hbm.at[pl.ds(c * N + NSUB * RQ, TAIL)])


def _deg_call(dst_f, ones, zeros):
    fn = pl.kernel(
        _deg_body,
        out_type=jax.ShapeDtypeStruct((2 * N, DH), jnp.float32),
        mesh=_mesh(),
        scratch_types=[
            pltpu.VMEM((NCH, CH), jnp.int32),
            pltpu.VMEM((CH, DH), jnp.float32),
            pltpu.VMEM_SHARED((N, DH), jnp.float32),
        ],
    )
    return fn(dst_f, ones, zeros)


def _agg_call(y, srcw, dst_r):
    fn = pl.kernel(
        _agg_body,
        out_type=jax.ShapeDtypeStruct((2 * N, DH), jnp.float32),
        mesh=_mesh(),
        scratch_types=[
            pltpu.VMEM((GP, CH), jnp.int32),
            pltpu.VMEM((GP, CH), jnp.int32),
            pltpu.VMEM((CH, DH), jnp.float32),
            pltpu.VMEM((CH, DH), jnp.float32),
            pltpu.VMEM_SHARED((N, DH), jnp.float32),
            pltpu.SemaphoreType.DMA,
            pltpu.SemaphoreType.DMA,
        ],
    )
    return fn(y, srcw, dst_r)


def _dis_block(p0_ref, p1_ref):
    deg = 1.0 + p0_ref[:, :1] + p1_ref[:, :1]
    return lax.rsqrt(deg)


def _tca_body(x_ref, w_ref, o_ref):
    o_ref[...] = jnp.dot(x_ref[...], w_ref[...],
                         preferred_element_type=jnp.float32)


def _scale_body(u_ref, p0_ref, p1_ref, o_ref):
    o_ref[...] = u_ref[...] * _dis_block(p0_ref, p1_ref)


def _tcb_body(al_ref, ah_ref, p0_ref, p1_ref, w_ref, b_ref, o_ref):
    dis = _dis_block(p0_ref, p1_ref)
    agg = jnp.concatenate([al_ref[...], ah_ref[...]], axis=1)
    h = jnp.maximum(agg * dis + b_ref[...], 0.0)
    y = jnp.dot(h, w_ref[...], preferred_element_type=jnp.float32)
    o_ref[...] = y * dis


def _tcc_body(al_ref, ah_ref, p0_ref, p1_ref, b2_ref, wp1_ref, bp1_ref,
              wp2_ref, bp2_ref, o_ref):
    dis = _dis_block(p0_ref, p1_ref)
    agg = jnp.concatenate([al_ref[...], ah_ref[...]], axis=1)
    h = jnp.maximum(agg * dis + b2_ref[...], 0.0)
    t = jnp.dot(h, wp1_ref[...], preferred_element_type=jnp.float32) + bp1_ref[...]
    o = jnp.dot(t, wp2_ref[...], preferred_element_type=jnp.float32) + bp2_ref[...]
    m = jnp.max(o, axis=1, keepdims=True)
    ex = jnp.exp(o - m)
    ssum = jnp.sum(ex, axis=1, keepdims=True)
    o_ref[...] = (o - m) - jnp.log(ssum)


def _tca_call(x, W1):
    return pl.pallas_call(
        _tca_body,
        grid=(NB, 2),
        in_specs=[
            pl.BlockSpec((BR, D), lambda i, h: (i, 0)),
            pl.BlockSpec((D, DH), lambda i, h: (0, h)),
        ],
        out_specs=pl.BlockSpec((BR, DH), lambda i, h: (i + h * NB, 0)),
        out_shape=jax.ShapeDtypeStruct((2 * N, DH), jnp.float32),
    )(x, W1)


def _scale_call(u, p8):
    return pl.pallas_call(
        _scale_body,
        grid=(2 * NB,),
        in_specs=[
            pl.BlockSpec((BR, DH), lambda i: (i, 0)),
            pl.BlockSpec((BR, 8), lambda i: (i % NB, 0)),
            pl.BlockSpec((BR, 8), lambda i: (i % NB + NB, 0)),
        ],
        out_specs=pl.BlockSpec((BR, DH), lambda i: (i, 0)),
        out_shape=jax.ShapeDtypeStruct((2 * N, DH), jnp.float32),
    )(u, p8, p8)


def _tcb_call(agg, p8, W2, b1):
    return pl.pallas_call(
        _tcb_body,
        grid=(NB, 2),
        in_specs=[
            pl.BlockSpec((BR, DH), lambda i, h: (i, 0)),
            pl.BlockSpec((BR, DH), lambda i, h: (i + NB, 0)),
            pl.BlockSpec((BR, 8), lambda i, h: (i, 0)),
            pl.BlockSpec((BR, 8), lambda i, h: (i + NB, 0)),
            pl.BlockSpec((D, DH), lambda i, h: (0, h)),
            pl.BlockSpec((1, D), lambda i, h: (0, 0)),
        ],
        out_specs=pl.BlockSpec((BR, DH), lambda i, h: (i + h * NB, 0)),
        out_shape=jax.ShapeDtypeStruct((2 * N, DH), jnp.float32),
    )(agg, agg, p8, p8, W2, b1)


def _tcc_call(agg, p8, b2, Wp1, bp1, Wp2, bp2):
    return pl.pallas_call(
        _tcc_body,
        grid=(NB,),
        in_specs=[
            pl.BlockSpec((BR, DH), lambda i: (i, 0)),
            pl.BlockSpec((BR, DH), lambda i: (i + NB, 0)),
            pl.BlockSpec((BR, 8), lambda i: (i, 0)),
            pl.BlockSpec((BR, 8), lambda i: (i + NB, 0)),
            pl.BlockSpec((1, D), lambda i: (0, 0)),
            pl.BlockSpec((D, D), lambda i: (0, 0)),
            pl.BlockSpec((1, D), lambda i: (0, 0)),
            pl.BlockSpec((D, D), lambda i: (0, 0)),
            pl.BlockSpec((1, D), lambda i: (0, 0)),
        ],
        out_specs=pl.BlockSpec((BR, D), lambda i: (i, 0)),
        out_shape=jax.ShapeDtypeStruct((N, D), jnp.float32),
    )(agg, agg, p8, p8, b2, Wp1, bp1, Wp2, bp2)


def kernel(x, edge_index, W1, b1, W2, b2, Wp1, bp1, Wp2, bp2):
    ei = edge_index.astype(jnp.int32)
    src = ei[0]
    dst = ei[1]
    srcw = jnp.stack([src, src + N]).reshape(2, NSUB, NGP, GP, CH)
    dst_r = dst.reshape(NSUB, NGP, GP, CH)
    dst_f = dst.reshape(NSUB, NCH, CH)
    ones = jnp.ones((CH, DH), jnp.float32)
    zeros = jnp.zeros((RQ, DH), jnp.float32)

    degs = _deg_call(dst_f, ones, zeros)
    p8 = degs[:, :8]
    xwr = _tca_call(x, W1)
    y1 = _scale_call(xwr, p8)
    agg1 = _agg_call(y1, srcw, dst_r)
    y2 = _tcb_call(agg1, p8, W2, b1.reshape(1, D))
    agg2 = _agg_call(y2, srcw, dst_r)
    return _tcc_call(agg2, p8, b2.reshape(1, D), Wp1, bp1.reshape(1, D),
                     Wp2, bp2.reshape(1, D))
